# baseline (speedup 1.0000x reference)
import functools

import jax
import jax.numpy as jnp
from jax import lax
from jax.experimental import pallas as pl
from jax.experimental.pallas import tpu as pltpu
from jax.experimental.pallas import tpu_sc as plsc

_LANES = 128


def _fmt_body(D, yt_ref, xt_ref, o_ref):
    o_ref[:, :D] = jnp.swapaxes(yt_ref[...], 0, 1)
    o_ref[:, D:] = jnp.swapaxes(xt_ref[...], 0, 1)


@functools.lru_cache(maxsize=None)
def _make_fmt(V, D):
    R = 6400
    return pl.pallas_call(
        functools.partial(_fmt_body, D),
        grid=(pl.cdiv(V, R),),
        in_specs=[
            pl.BlockSpec((D, R), lambda s: (0, s)),
            pl.BlockSpec((D, R), lambda s: (0, s)),
        ],
        out_specs=pl.BlockSpec((R, _LANES), lambda s: (s, 0)),
        out_shape=jax.ShapeDtypeStruct((V, _LANES), jnp.float32),
    )


@functools.lru_cache(maxsize=None)
def _make_sc_gather(B, D):
    info = plsc.get_sparse_core_info()
    NC, NS = info.num_cores, info.num_subcores
    NW = NC * NS
    assert B % (8 * NW) == 0
    b_per_w = B // NW
    mesh = plsc.VectorSubcoreMesh(core_axis_name="c", subcore_axis_name="s")

    @functools.partial(
        pl.kernel,
        mesh=mesh,
        compiler_params=pltpu.CompilerParams(use_tc_tiling_on_sc=False),
        out_type=jax.ShapeDtypeStruct((B, _LANES), jnp.float32),
        scratch_types=[
            pltpu.VMEM((b_per_w,), jnp.int32),
            pltpu.VMEM((b_per_w,), jnp.int32),
            pltpu.VMEM((b_per_w, D), jnp.float32),
            pltpu.VMEM((b_per_w, D), jnp.float32),
            pltpu.SemaphoreType.DMA,
            pltpu.SemaphoreType.DMA,
        ],
    )
    def sc_gather(yi_hbm, xi_hbm, tab_hbm, z_hbm,
                  yi_v, xi_v, yrows_v, xrows_v, semy, semx):
        wid = lax.axis_index("s") * NC + lax.axis_index("c")
        base = wid * b_per_w
        pltpu.sync_copy(yi_hbm.at[pl.ds(base, b_per_w)], yi_v)
        pltpu.sync_copy(xi_hbm.at[pl.ds(base, b_per_w)], xi_v)
        for c in range(b_per_w // 16):
            sl = pl.ds(c * 16, 16)
            yi_v[sl] = yi_v[sl] * 2
            xi_v[sl] = xi_v[sl] * 2 + 1
        cy = pltpu.async_copy(tab_hbm.at[yi_v], yrows_v, semy)
        cx = pltpu.async_copy(tab_hbm.at[xi_v], xrows_v, semx)
        cy.wait()
        cx.wait()
        pltpu.sync_copy(yrows_v,
                        z_hbm.at[pl.ds(base, b_per_w), pl.ds(0, D)])
        pltpu.sync_copy(xrows_v,
                        z_hbm.at[pl.ds(base, b_per_w), pl.ds(D, D)])

    return sc_gather


def _outer_sum_body(S, D, z_ref, o_ref, xt_scr):
    i = pl.program_id(0)

    @pl.when(i == 0)
    def _prologue():
        for j in range(S):
            xt_scr[j] = jnp.swapaxes(z_ref[j][:, D:], 0, 1)

    yt = jnp.swapaxes(z_ref[i][:, :D], 0, 1)
    for j in range(S):
        o_ref[0, j] = yt + xt_scr[j]


@functools.lru_cache(maxsize=None)
def _make_outer_sum(N, S, D):
    return pl.pallas_call(
        functools.partial(_outer_sum_body, S, D),
        grid=(S,),
        in_specs=[
            pl.BlockSpec((S, N, _LANES), lambda i: (0, 0, 0)),
        ],
        out_specs=pl.BlockSpec((1, S, D, N), lambda i: (i, 0, 0, 0)),
        out_shape=jax.ShapeDtypeStruct((S, S, D, N), jnp.float32),
        scratch_shapes=[pltpu.VMEM((S, D, N), jnp.float32)],
    )


def kernel(y_indexes, x_indexes, x_table, y_table):
    N, S = x_indexes.shape
    V, D = x_table.shape
    B = N * S

    yi = y_indexes.T.reshape(B).astype(jnp.int32)
    xi = x_indexes.T.reshape(B).astype(jnp.int32)

    tab_w = _make_fmt(V, D)(
        jnp.swapaxes(y_table, 0, 1), jnp.swapaxes(x_table, 0, 1))

    z = _make_sc_gather(B, D)(yi, xi, tab_w.reshape(2 * V, D))

    out_phys = _make_outer_sum(N, S, D)(z.reshape(S, N, _LANES))
    return jnp.transpose(out_phys, (3, 2, 0, 1))


# speedup vs baseline: 1.0200x; 1.0200x over previous
import functools

import jax
import jax.numpy as jnp
from jax import lax
from jax.experimental import pallas as pl
from jax.experimental.pallas import tpu as pltpu
from jax.experimental.pallas import tpu_sc as plsc

_LANES = 128


def _fmt_body(D, yt_ref, xt_ref, o_ref):
    o_ref[:, :D] = jnp.swapaxes(yt_ref[...], 0, 1)
    o_ref[:, D:] = jnp.swapaxes(xt_ref[...], 0, 1)


@functools.lru_cache(maxsize=None)
def _make_fmt(V, D):
    R = 12800
    return pl.pallas_call(
        functools.partial(_fmt_body, D),
        grid=(pl.cdiv(V, R),),
        in_specs=[
            pl.BlockSpec((D, R), lambda s: (0, s)),
            pl.BlockSpec((D, R), lambda s: (0, s)),
        ],
        out_specs=pl.BlockSpec((R, _LANES), lambda s: (s, 0)),
        out_shape=jax.ShapeDtypeStruct((V, _LANES), jnp.float32),
    )


@functools.lru_cache(maxsize=None)
def _make_sc_gather(B, D):
    info = plsc.get_sparse_core_info()
    NC, NS = info.num_cores, info.num_subcores
    NW = NC * NS
    assert B % (8 * NW) == 0
    b_per_w = B // NW
    mesh = plsc.VectorSubcoreMesh(core_axis_name="c", subcore_axis_name="s")

    @functools.partial(
        pl.kernel,
        mesh=mesh,
        compiler_params=pltpu.CompilerParams(use_tc_tiling_on_sc=False),
        out_type=jax.ShapeDtypeStruct((B, _LANES), jnp.float32),
        scratch_types=[
            pltpu.VMEM((b_per_w,), jnp.int32),
            pltpu.VMEM((b_per_w,), jnp.int32),
            pltpu.VMEM((b_per_w, D), jnp.float32),
            pltpu.VMEM((b_per_w, D), jnp.float32),
            pltpu.SemaphoreType.DMA,
            pltpu.SemaphoreType.DMA,
        ],
    )
    def sc_gather(yi_hbm, xi_hbm, tab_hbm, z_hbm,
                  yi_v, xi_v, yrows_v, xrows_v, semy, semx):
        wid = lax.axis_index("s") * NC + lax.axis_index("c")
        base = wid * b_per_w
        pltpu.sync_copy(yi_hbm.at[pl.ds(base, b_per_w)], yi_v)
        pltpu.sync_copy(xi_hbm.at[pl.ds(base, b_per_w)], xi_v)
        for c in range(b_per_w // 16):
            sl = pl.ds(c * 16, 16)
            yi_v[sl] = yi_v[sl] * 2
            xi_v[sl] = xi_v[sl] * 2 + 1
        cy = pltpu.async_copy(tab_hbm.at[yi_v], yrows_v, semy)
        cx = pltpu.async_copy(tab_hbm.at[xi_v], xrows_v, semx)
        cy.wait()
        cx.wait()
        pltpu.sync_copy(yrows_v,
                        z_hbm.at[pl.ds(base, b_per_w), pl.ds(0, D)])
        pltpu.sync_copy(xrows_v,
                        z_hbm.at[pl.ds(base, b_per_w), pl.ds(D, D)])

    return sc_gather


def _outer_sum_body(S, D, z_ref, o_ref, xt_scr):
    i = pl.program_id(0)

    @pl.when(i == 0)
    def _prologue():
        for j in range(S):
            xt_scr[j] = jnp.swapaxes(z_ref[j][:, D:], 0, 1)

    yt = jnp.swapaxes(z_ref[i][:, :D], 0, 1)
    for j in range(S):
        o_ref[0, j] = yt + xt_scr[j]


@functools.lru_cache(maxsize=None)
def _make_outer_sum(N, S, D):
    return pl.pallas_call(
        functools.partial(_outer_sum_body, S, D),
        grid=(S,),
        in_specs=[
            pl.BlockSpec((S, N, _LANES), lambda i: (0, 0, 0)),
        ],
        out_specs=pl.BlockSpec((1, S, D, N), lambda i: (i, 0, 0, 0)),
        out_shape=jax.ShapeDtypeStruct((S, S, D, N), jnp.float32),
        scratch_shapes=[pltpu.VMEM((S, D, N), jnp.float32)],
    )


def kernel(y_indexes, x_indexes, x_table, y_table):
    N, S = x_indexes.shape
    V, D = x_table.shape
    B = N * S

    yi = y_indexes.T.reshape(B).astype(jnp.int32)
    xi = x_indexes.T.reshape(B).astype(jnp.int32)

    tab_w = _make_fmt(V, D)(
        jnp.swapaxes(y_table, 0, 1), jnp.swapaxes(x_table, 0, 1))

    z = _make_sc_gather(B, D)(yi, xi, tab_w.reshape(2 * V, D))

    out_phys = _make_outer_sum(N, S, D)(z.reshape(S, N, _LANES))
    return jnp.transpose(out_phys, (3, 2, 0, 1))
